# baseline (speedup 1.0000x reference)
import functools

import jax
import jax.numpy as jnp
from jax import lax
from jax.experimental import pallas as pl
from jax.experimental.pallas import tpu as pltpu
from jax.experimental.pallas import tpu_sc as plsc

BATCH = 16384
EMBED_DIM = 128
NUM_CORES = 2
NUM_SUBCORES = 16
NUM_WORKERS = NUM_CORES * NUM_SUBCORES
ROWS_PER_WORKER = BATCH // NUM_WORKERS
CHUNK = 128
NUM_CHUNKS = ROWS_PER_WORKER // CHUNK
LANES = 16


def _distmult_body(head_hbm, idx_hbm, tail_hbm, rel_hbm, out_hbm,
                   idx_v, out_v, h_v, t_v, r_v, sem):
    wid = lax.axis_index("s") * NUM_CORES + lax.axis_index("c")
    base = wid * ROWS_PER_WORKER
    lane_iota = lax.iota(jnp.int32, LANES)

    pltpu.sync_copy(idx_hbm.at[pl.ds(base, ROWS_PER_WORKER)], idx_v)

    def chunk_body(ci, carry):
        cbase = base + ci * CHUNK
        idx_slice = idx_v.at[pl.ds(ci * CHUNK, CHUNK)]
        gather = pltpu.async_copy(rel_hbm.at[idx_slice], r_v, sem)
        pltpu.sync_copy(head_hbm.at[pl.ds(cbase, CHUNK)], h_v)
        pltpu.sync_copy(tail_hbm.at[pl.ds(cbase, CHUNK)], t_v)
        gather.wait()

        def group_body(g, carry2):
            rows = g * LANES + lane_iota

            def d_body(dd, acc):
                dvec = jnp.full((LANES,), 0, jnp.int32) + dd
                hh = plsc.load_gather(h_v, [rows, dvec])
                rr = plsc.load_gather(r_v, [rows, dvec])
                tt = plsc.load_gather(t_v, [rows, dvec])
                return acc + (hh * rr) * tt

            acc = lax.fori_loop(0, EMBED_DIM, d_body,
                                jnp.zeros((LANES,), jnp.float32),
                                unroll=4)
            out_v[pl.ds(ci * CHUNK + g * LANES, LANES)] = acc
            return carry2

        lax.fori_loop(0, CHUNK // LANES, group_body, 0)
        return carry

    lax.fori_loop(0, NUM_CHUNKS, chunk_body, 0)
    pltpu.sync_copy(out_v, out_hbm.at[pl.ds(base, ROWS_PER_WORKER)])


@jax.jit
def _distmult_sc(head_e, rel_idx, tail_e, rel_embedding):
    mesh = plsc.VectorSubcoreMesh(core_axis_name="c", subcore_axis_name="s")
    kern = functools.partial(
        pl.kernel,
        mesh=mesh,
        compiler_params=pltpu.CompilerParams(needs_layout_passes=False),
        out_type=jax.ShapeDtypeStruct((BATCH,), jnp.float32),
        scratch_types=[
            pltpu.VMEM((ROWS_PER_WORKER,), jnp.int32),
            pltpu.VMEM((ROWS_PER_WORKER,), jnp.float32),
            pltpu.VMEM((CHUNK, EMBED_DIM), jnp.float32),
            pltpu.VMEM((CHUNK, EMBED_DIM), jnp.float32),
            pltpu.VMEM((CHUNK, EMBED_DIM), jnp.float32),
            pltpu.SemaphoreType.DMA,
        ],
    )(_distmult_body)
    return kern(head_e, rel_idx, tail_e, rel_embedding)


def kernel(head_e, rel_idx, tail_e, rel_embedding):
    return _distmult_sc(head_e, rel_idx.astype(jnp.int32), tail_e,
                        rel_embedding)


# speedup vs baseline: 3.0876x; 3.0876x over previous
import functools

import jax
import jax.numpy as jnp
from jax import lax
from jax.experimental import pallas as pl
from jax.experimental.pallas import tpu as pltpu
from jax.experimental.pallas import tpu_sc as plsc

BATCH = 16384
EMBED_DIM = 128
NUM_CORES = 2
NUM_SUBCORES = 16
NUM_WORKERS = NUM_CORES * NUM_SUBCORES
ROWS_PER_WORKER = BATCH // NUM_WORKERS
CHUNK = 128
NUM_CHUNKS = ROWS_PER_WORKER // CHUNK
LANES = 16


def _distmult_body(head_hbm, idx_hbm, tail_hbm, rel_hbm, out_hbm,
                   idx_v, out_v, h_v, t_v, r_v, sem):
    wid = lax.axis_index("s") * NUM_CORES + lax.axis_index("c")
    base = wid * ROWS_PER_WORKER
    lane_iota = lax.iota(jnp.int32, LANES)

    pltpu.sync_copy(idx_hbm.at[pl.ds(base, ROWS_PER_WORKER)], idx_v)

    def chunk_body(ci, carry):
        cbase = base + ci * CHUNK
        idx_slice = idx_v.at[pl.ds(ci * CHUNK, CHUNK)]
        gather = pltpu.async_copy(rel_hbm.at[idx_slice], r_v, sem)
        pltpu.sync_copy(head_hbm.at[pl.ds(cbase, CHUNK)], h_v)
        pltpu.sync_copy(tail_hbm.at[pl.ds(cbase, CHUNK)], t_v)
        gather.wait()

        def group_body(g, carry2):
            rows = g * LANES + lane_iota

            def d_body(dd, carry3):
                acc, dvec = carry3
                hh = plsc.load_gather(h_v, [rows, dvec])
                rr = plsc.load_gather(r_v, [rows, dvec])
                tt = plsc.load_gather(t_v, [rows, dvec])
                return acc + (hh * rr) * tt, (dvec + 1) & (EMBED_DIM - 1)

            acc, _ = lax.fori_loop(
                0, EMBED_DIM, d_body,
                (jnp.zeros((LANES,), jnp.float32), lane_iota),
                unroll=8)
            out_v[pl.ds(ci * CHUNK + g * LANES, LANES)] = acc
            return carry2

        lax.fori_loop(0, CHUNK // LANES, group_body, 0)
        return carry

    lax.fori_loop(0, NUM_CHUNKS, chunk_body, 0)
    pltpu.sync_copy(out_v, out_hbm.at[pl.ds(base, ROWS_PER_WORKER)])


@jax.jit
def _distmult_sc(head_e, rel_idx, tail_e, rel_embedding):
    mesh = plsc.VectorSubcoreMesh(core_axis_name="c", subcore_axis_name="s")
    kern = functools.partial(
        pl.kernel,
        mesh=mesh,
        compiler_params=pltpu.CompilerParams(needs_layout_passes=False),
        out_type=jax.ShapeDtypeStruct((BATCH,), jnp.float32),
        scratch_types=[
            pltpu.VMEM((ROWS_PER_WORKER,), jnp.int32),
            pltpu.VMEM((ROWS_PER_WORKER,), jnp.float32),
            pltpu.VMEM((CHUNK, EMBED_DIM), jnp.float32),
            pltpu.VMEM((CHUNK, EMBED_DIM), jnp.float32),
            pltpu.VMEM((CHUNK, EMBED_DIM), jnp.float32),
            pltpu.SemaphoreType.DMA,
        ],
    )(_distmult_body)
    return kern(head_e, rel_idx, tail_e, rel_embedding)


def kernel(head_e, rel_idx, tail_e, rel_embedding):
    return _distmult_sc(head_e, rel_idx.astype(jnp.int32), tail_e,
                        rel_embedding)


# speedup vs baseline: 3.5713x; 1.1567x over previous
import functools

import jax
import jax.numpy as jnp
from jax import lax
from jax.experimental import pallas as pl
from jax.experimental.pallas import tpu as pltpu
from jax.experimental.pallas import tpu_sc as plsc

BATCH = 16384
EMBED_DIM = 128
NUM_CORES = 2
NUM_SUBCORES = 16
NUM_WORKERS = NUM_CORES * NUM_SUBCORES
ROWS_PER_WORKER = BATCH // NUM_WORKERS
CHUNK = 128
NUM_CHUNKS = ROWS_PER_WORKER // CHUNK
NBUF = 2
LANES = 16


def _distmult_body(head_hbm, idx_hbm, tail_hbm, rel_hbm, out_hbm,
                   idx_v, out_v, h_v, t_v, r_v, sems):
    wid = lax.axis_index("s") * NUM_CORES + lax.axis_index("c")
    base = wid * ROWS_PER_WORKER
    lane_iota = lax.iota(jnp.int32, LANES)

    pltpu.sync_copy(idx_hbm.at[pl.ds(base, ROWS_PER_WORKER)], idx_v)

    def fire(ci):
        b = ci % NBUF
        cbase = base + ci * CHUNK
        idx_slice = idx_v.at[pl.ds(ci * CHUNK, CHUNK)]
        return (
            pltpu.async_copy(rel_hbm.at[idx_slice], r_v.at[b], sems.at[b]),
            pltpu.async_copy(head_hbm.at[pl.ds(cbase, CHUNK)], h_v.at[b],
                             sems.at[b]),
            pltpu.async_copy(tail_hbm.at[pl.ds(cbase, CHUNK)], t_v.at[b],
                             sems.at[b]),
        )

    def compute(ci):
        b = ci % NBUF
        bvec = jnp.full((LANES,), b, jnp.int32)

        def group_body(g, carry2):
            rows = g * LANES + lane_iota

            def d_body(dd, carry3):
                acc, dvec = carry3
                hh = plsc.load_gather(h_v, [bvec, rows, dvec])
                rr = plsc.load_gather(r_v, [bvec, rows, dvec])
                tt = plsc.load_gather(t_v, [bvec, rows, dvec])
                return acc + (hh * rr) * tt, (dvec + 1) & (EMBED_DIM - 1)

            acc, _ = lax.fori_loop(
                0, EMBED_DIM, d_body,
                (jnp.zeros((LANES,), jnp.float32), lane_iota),
                unroll=8)
            out_v[pl.ds(ci * CHUNK + g * LANES, LANES)] = acc
            return carry2

        lax.fori_loop(0, CHUNK // LANES, group_body, 0)

    copies = fire(0)
    for ci in range(NUM_CHUNKS):
        nxt = fire(ci + 1) if ci + 1 < NUM_CHUNKS else ()
        for c in copies:
            c.wait()
        compute(ci)
        copies = nxt

    pltpu.sync_copy(out_v, out_hbm.at[pl.ds(base, ROWS_PER_WORKER)])


@jax.jit
def _distmult_sc(head_e, rel_idx, tail_e, rel_embedding):
    mesh = plsc.VectorSubcoreMesh(core_axis_name="c", subcore_axis_name="s")
    kern = functools.partial(
        pl.kernel,
        mesh=mesh,
        compiler_params=pltpu.CompilerParams(needs_layout_passes=False),
        out_type=jax.ShapeDtypeStruct((BATCH,), jnp.float32),
        scratch_types=[
            pltpu.VMEM((ROWS_PER_WORKER,), jnp.int32),
            pltpu.VMEM((ROWS_PER_WORKER,), jnp.float32),
            pltpu.VMEM((NBUF, CHUNK, EMBED_DIM), jnp.float32),
            pltpu.VMEM((NBUF, CHUNK, EMBED_DIM), jnp.float32),
            pltpu.VMEM((NBUF, CHUNK, EMBED_DIM), jnp.float32),
            pltpu.SemaphoreType.DMA((NBUF,)),
        ],
    )(_distmult_body)
    return kern(head_e, rel_idx, tail_e, rel_embedding)


def kernel(head_e, rel_idx, tail_e, rel_embedding):
    return _distmult_sc(head_e, rel_idx.astype(jnp.int32), tail_e,
                        rel_embedding)


# speedup vs baseline: 3.5729x; 1.0004x over previous
import functools

import jax
import jax.numpy as jnp
from jax import lax
from jax.experimental import pallas as pl
from jax.experimental.pallas import tpu as pltpu
from jax.experimental.pallas import tpu_sc as plsc

BATCH = 16384
EMBED_DIM = 128
NUM_CORES = 2
NUM_SUBCORES = 16
NUM_WORKERS = NUM_CORES * NUM_SUBCORES
ROWS_PER_WORKER = BATCH // NUM_WORKERS
CHUNK = 128
NUM_CHUNKS = ROWS_PER_WORKER // CHUNK
NBUF = 2
LANES = 16


def _distmult_body(head_hbm, idx_hbm, tail_hbm, rel_hbm, out_hbm,
                   idx_v, out_v, h_v, t_v, r_v, sems):
    wid = lax.axis_index("s") * NUM_CORES + lax.axis_index("c")
    base = wid * ROWS_PER_WORKER
    lane_iota = lax.iota(jnp.int32, LANES)

    pltpu.sync_copy(idx_hbm.at[pl.ds(base, ROWS_PER_WORKER)], idx_v)

    def fire(ci):
        b = ci % NBUF
        cbase = base + ci * CHUNK
        idx_slice = idx_v.at[pl.ds(ci * CHUNK, CHUNK)]
        return (
            pltpu.async_copy(rel_hbm.at[idx_slice], r_v.at[b], sems.at[b]),
            pltpu.async_copy(head_hbm.at[pl.ds(cbase, CHUNK)], h_v.at[b],
                             sems.at[b]),
            pltpu.async_copy(tail_hbm.at[pl.ds(cbase, CHUNK)], t_v.at[b],
                             sems.at[b]),
        )

    def compute(ci):
        b = ci % NBUF
        bvec = jnp.full((LANES,), b, jnp.int32)

        def group_body(g, carry2):
            rows = g * LANES + lane_iota

            def d_body(dd, carry3):
                acc, dvec = carry3
                hh = plsc.load_gather(h_v, [bvec, rows, dvec])
                rr = plsc.load_gather(r_v, [bvec, rows, dvec])
                tt = plsc.load_gather(t_v, [bvec, rows, dvec])
                return acc + (hh * rr) * tt, (dvec + 1) & (EMBED_DIM - 1)

            acc, _ = lax.fori_loop(
                0, EMBED_DIM, d_body,
                (jnp.zeros((LANES,), jnp.float32), lane_iota),
                unroll=8)
            out_v[pl.ds(ci * CHUNK + g * LANES, LANES)] = acc
            return carry2

        lax.fori_loop(0, CHUNK // LANES, group_body, 0)

    copies = fire(0)
    for ci in range(NUM_CHUNKS):
        nxt = fire(ci + 1) if ci + 1 < NUM_CHUNKS else ()
        for c in copies:
            c.wait()
        compute(ci)
        copies = nxt

    pltpu.sync_copy(out_v, out_hbm.at[pl.ds(base, ROWS_PER_WORKER)])


@jax.jit
def _distmult_sc(head_e, rel_idx, tail_e, rel_embedding):
    mesh = plsc.VectorSubcoreMesh(core_axis_name="c", subcore_axis_name="s")
    kern = functools.partial(
        pl.kernel,
        mesh=mesh,
        compiler_params=pltpu.CompilerParams(needs_layout_passes=False,
                                             skip_device_barrier=True),
        out_type=jax.ShapeDtypeStruct((BATCH,), jnp.float32),
        scratch_types=[
            pltpu.VMEM((ROWS_PER_WORKER,), jnp.int32),
            pltpu.VMEM((ROWS_PER_WORKER,), jnp.float32),
            pltpu.VMEM((NBUF, CHUNK, EMBED_DIM), jnp.float32),
            pltpu.VMEM((NBUF, CHUNK, EMBED_DIM), jnp.float32),
            pltpu.VMEM((NBUF, CHUNK, EMBED_DIM), jnp.float32),
            pltpu.SemaphoreType.DMA((NBUF,)),
        ],
    )(_distmult_body)
    return kern(head_e, rel_idx, tail_e, rel_embedding)


def kernel(head_e, rel_idx, tail_e, rel_embedding):
    return _distmult_sc(head_e, rel_idx.astype(jnp.int32), tail_e,
                        rel_embedding)
